# baseline (speedup 1.0000x reference)
import functools

import jax
import jax.numpy as jnp
from jax import lax
from jax.experimental import pallas as pl
from jax.experimental.pallas import tpu as pltpu


def _dot(a, b):
    return lax.dot_general(
        a, b, (((1,), (0,)), ((), ())), preferred_element_type=jnp.float32
    )


def _dot_t(a, b):
    return lax.dot_general(
        a, b, (((1,), (1,)), ((), ())), preferred_element_type=jnp.float32
    )


def _full(shape):
    return pl.BlockSpec(shape, lambda i: (0,) * len(shape))


def _make_k1_body(n_prep, n_seq, ar):
    def body(go_ref, seq_ref, adj_ref, mw1_ref, mb1_ref, mw2_ref, mb2_ref,
             gw1_ref, g1b_ref, gw2_ref, f1w_ref, f1b_ref, f2w_ref, f2b_ref,
             hsem_ref, hsem16_ref, seqout16_ref, s2_ref, s1_ref):
        i = pl.program_id(0)

        @pl.when(i < n_prep)
        def _go_prep():
            g = go_ref[...]
            h = jnp.maximum(_dot(g, mw1_ref[...]) + mb1_ref[...], 0.0)
            hsem = _dot(h, mw2_ref[...]) + mb2_ref[...]
            hsem_ref[...] = hsem
            hsem16_ref[...] = hsem.astype(jnp.bfloat16)
            gr = go_ref.shape[0]
            s1_ref[pl.ds(i * gr, gr), :] = _dot(g, gw1_ref[...])

        @pl.when(i < n_seq)
        def _seq_prep():
            h = jnp.maximum(_dot(seq_ref[...], f1w_ref[...]) + f1b_ref[...],
                            0.0)
            so = _dot(h, f2w_ref[...]) + f2b_ref[...]
            seqout16_ref[...] = so.astype(jnp.bfloat16)

        @pl.when(i >= n_prep)
        def _phase1():
            x = jnp.maximum(_dot(adj_ref[...], s1_ref[...]) + g1b_ref[...],
                            0.0)
            s2_ref[...] = _dot(x, gw2_ref[...])

    return body


def _k2_body(adj_ref, s2_ref, g2b_ref, hsem16_ref, seqout16_ref,
             hstruct_ref, pred_ref):
    hs = jnp.maximum(_dot(adj_ref[...], s2_ref[...]) + g2b_ref[...], 0.0)
    hstruct_ref[...] = hs
    go_blk = jnp.concatenate(
        [hsem16_ref[...], hs.astype(jnp.bfloat16)], axis=1)
    pred_ref[...] = jax.nn.sigmoid(_dot_t(seqout16_ref[...], go_blk))


def kernel(seq_embed, go_embed, adj, mlp_w1, mlp_b1, mlp_w2, mlp_b2,
           gc1_w, gc1_b, gc2_w, gc2_b, fc1_w, fc1_b, fc2_w, fc2_b):
    N, _ = adj.shape
    B, d_seq = seq_embed.shape
    go_feat = go_embed.shape[1]
    h0 = mlp_w1.shape[1]
    h1 = mlp_w2.shape[1]

    mb1 = mlp_b1.reshape(1, h0)
    mb2 = mlp_b2.reshape(1, h1)
    g1b = gc1_b.reshape(1, h0)
    g2b = gc2_b.reshape(1, h1)
    f1b = fc1_b.reshape(1, h0)
    f2b = fc2_b.reshape(1, 2 * h1)

    gr = min(1024, N)
    sr = min(1024, B)
    ar = min(512, N)
    n_prep = N // gr
    n_seq = B // sr
    nb = N // ar
    g1 = n_prep + nb

    h_semantic, hsem16, seqout16, s2 = pl.pallas_call(
        _make_k1_body(n_prep, n_seq, ar),
        grid=(g1,),
        in_specs=[
            pl.BlockSpec((gr, go_feat), lambda i: (lax.min(i, n_prep - 1), 0)),
            pl.BlockSpec((sr, d_seq), lambda i: (lax.min(i, n_seq - 1), 0)),
            pl.BlockSpec((ar, N), lambda i: (lax.max(i - n_prep, 0), 0)),
            _full((go_feat, h0)), _full((1, h0)),
            _full((h0, h1)), _full((1, h1)),
            _full((go_feat, h0)), _full((1, h0)),
            _full((h0, h1)),
            _full((d_seq, h0)), _full((1, h0)),
            _full((h0, 2 * h1)), _full((1, 2 * h1)),
        ],
        out_specs=[
            pl.BlockSpec((gr, h1), lambda i: (lax.min(i, n_prep - 1), 0)),
            pl.BlockSpec((gr, h1), lambda i: (lax.min(i, n_prep - 1), 0)),
            pl.BlockSpec((sr, 2 * h1), lambda i: (lax.min(i, n_seq - 1), 0)),
            pl.BlockSpec((ar, h1), lambda i: (lax.max(i - n_prep, 0), 0)),
        ],
        out_shape=[
            jax.ShapeDtypeStruct((N, h1), jnp.float32),
            jax.ShapeDtypeStruct((N, h1), jnp.bfloat16),
            jax.ShapeDtypeStruct((B, 2 * h1), jnp.bfloat16),
            jax.ShapeDtypeStruct((N, h1), jnp.float32),
        ],
        scratch_shapes=[
            pltpu.VMEM((N, h0), jnp.float32),
        ],
    )(go_embed, seq_embed, adj, mlp_w1, mb1, mlp_w2, mb2, gc1_w, g1b,
      gc2_w, fc1_w, f1b, fc2_w, f2b)

    h_structure, pred = pl.pallas_call(
        _k2_body,
        grid=(nb,),
        in_specs=[
            pl.BlockSpec((ar, N), lambda i: (i, 0)),
            _full((N, h1)), _full((1, h1)),
            pl.BlockSpec((ar, h1), lambda i: (i, 0)),
            _full((B, 2 * h1)),
        ],
        out_specs=[
            pl.BlockSpec((ar, h1), lambda i: (i, 0)),
            pl.BlockSpec((B, ar), lambda i: (0, i)),
        ],
        out_shape=[
            jax.ShapeDtypeStruct((N, h1), jnp.float32),
            jax.ShapeDtypeStruct((B, N), jnp.float32),
        ],
    )(adj, s2, g2b, hsem16, seqout16)

    return (h_semantic, h_structure, pred)
